# baseline (speedup 1.0000x reference)
import functools

import jax
import jax.numpy as jnp
from jax import lax
from jax.experimental import pallas as pl
from jax.experimental.pallas import tpu as pltpu
from jax.experimental.pallas import tpu_sc as plsc

N_ROWS = 8192
N_COLS = 2048
NC = 2
NS = 16
NW = NC * NS
ROWS_PER_W = N_ROWS // NW
CHUNK = 16
N_CHUNKS = ROWS_PER_W // CHUNK
LANES = 16
COL_VECS = N_COLS // LANES


def _permute_body(x_hbm, idx_hbm, out_hbm, idx_v, in_v, out_v):
    wid = lax.axis_index("s") * NC + lax.axis_index("c")
    row0 = wid * ROWS_PER_W

    pltpu.sync_copy(idx_hbm, idx_v)

    def chunk_body(ci, _):
        base = (row0 + ci * CHUNK) * N_COLS
        pltpu.sync_copy(x_hbm.at[pl.ds(base, CHUNK * N_COLS)], in_v)

        def col_body(j, _):
            j16 = pl.multiple_of(j * LANES, LANES)
            col_ids = idx_v[pl.ds(j16, LANES)]
            for r in range(CHUNK):
                v = plsc.load_gather(in_v, [col_ids + r * N_COLS])
                out_v[pl.ds(j16 + r * N_COLS, LANES)] = v
            return 0

        lax.fori_loop(0, COL_VECS, col_body, 0)
        pltpu.sync_copy(out_v, out_hbm.at[pl.ds(base, CHUNK * N_COLS)])
        return 0

    lax.fori_loop(0, N_CHUNKS, chunk_body, 0)


@jax.jit
def _permute(x_flat, idx):
    kern = pl.kernel(
        _permute_body,
        out_type=jax.ShapeDtypeStruct((N_ROWS * N_COLS,), jnp.float32),
        mesh=plsc.VectorSubcoreMesh(core_axis_name="c", subcore_axis_name="s"),
        scratch_types=[
            pltpu.VMEM((N_COLS,), jnp.int32),
            pltpu.VMEM((CHUNK * N_COLS,), jnp.float32),
            pltpu.VMEM((CHUNK * N_COLS,), jnp.float32),
        ],
        compiler_params=pltpu.CompilerParams(needs_layout_passes=False),
    )
    return kern(x_flat, idx)


def kernel(x, perm, perm_inv, rev):
    idx = jnp.where(rev, perm_inv, perm).astype(jnp.int32)
    out_flat = _permute(x.reshape(-1), idx)
    return out_flat.reshape(N_ROWS, N_COLS)


# speedup vs baseline: 1.1760x; 1.1760x over previous
import jax
import jax.numpy as jnp
from jax import lax
from jax.experimental import pallas as pl
from jax.experimental.pallas import tpu as pltpu
from jax.experimental.pallas import tpu_sc as plsc

N_ROWS = 8192
N_COLS = 2048
NC = 2
NS = 16
NW = NC * NS
ROWS_PER_W = N_ROWS // NW
CHUNK = 8
N_CHUNKS = ROWS_PER_W // CHUNK
N_PAIRS = N_CHUNKS // 2
LANES = 16
COL_VECS = N_COLS // LANES
CHUNK_VECS = CHUNK * COL_VECS
UNROLL = 4


def _permute_body(x_hbm, idx_hbm, out_hbm, idx_v, idx_all,
                  in_a, in_b, out_a, out_b,
                  sem_ia, sem_ib, sem_oa, sem_ob):
    wid = lax.axis_index("s") * NC + lax.axis_index("c")
    row0 = wid * ROWS_PER_W

    pltpu.sync_copy(idx_hbm, idx_v)

    for r in range(CHUNK):
        def build(j, _, r=r):
            j16 = pl.multiple_of(j * LANES, LANES)
            idx_all[pl.ds(r * N_COLS + j16, LANES)] = (
                idx_v[pl.ds(j16, LANES)] + r * N_COLS)
            return 0
        lax.fori_loop(0, COL_VECS, build, 0)

    def in_slice(ci):
        return x_hbm.at[pl.ds((row0 + ci * CHUNK) * N_COLS, CHUNK * N_COLS)]

    def out_slice(ci):
        return out_hbm.at[pl.ds((row0 + ci * CHUNK) * N_COLS, CHUNK * N_COLS)]

    def start_in(ci, buf, sem):
        pltpu.async_copy(in_slice(ci), buf, sem)

    def wait_in(ci, buf, sem):
        pltpu.make_async_copy(in_slice(ci), buf, sem).wait()

    def start_out(ci, buf, sem):
        pltpu.async_copy(buf, out_slice(ci), sem)

    def wait_out(ci, buf, sem):
        pltpu.make_async_copy(buf, out_slice(ci), sem).wait()

    def permute(in_v, out_v):
        def body(i, _):
            for u in range(UNROLL):
                p = pl.multiple_of((i * UNROLL + u) * LANES, LANES)
                ids = idx_all[pl.ds(p, LANES)]
                out_v[pl.ds(p, LANES)] = plsc.load_gather(in_v, [ids])
            return 0
        lax.fori_loop(0, CHUNK_VECS // UNROLL, body, 0)

    start_in(0, in_a, sem_ia)
    start_in(1, in_b, sem_ib)

    wait_in(0, in_a, sem_ia)
    permute(in_a, out_a)
    start_out(0, out_a, sem_oa)
    start_in(2, in_a, sem_ia)

    wait_in(1, in_b, sem_ib)
    permute(in_b, out_b)
    start_out(1, out_b, sem_ob)
    start_in(3, in_b, sem_ib)

    def pair_body(g, _):
        ci0 = 2 * g
        ci1 = ci0 + 1

        wait_in(ci0, in_a, sem_ia)
        wait_out(ci0, out_a, sem_oa)
        permute(in_a, out_a)
        start_out(ci0, out_a, sem_oa)
        start_in(jnp.minimum(ci0 + 2, N_CHUNKS - 1), in_a, sem_ia)

        wait_in(ci1, in_b, sem_ib)
        wait_out(ci1, out_b, sem_ob)
        permute(in_b, out_b)
        start_out(ci1, out_b, sem_ob)
        start_in(jnp.minimum(ci1 + 2, N_CHUNKS - 1), in_b, sem_ib)
        return 0

    lax.fori_loop(1, N_PAIRS, pair_body, 0)

    wait_in(N_CHUNKS - 1, in_a, sem_ia)
    wait_in(N_CHUNKS - 1, in_b, sem_ib)
    wait_out(N_CHUNKS - 2, out_a, sem_oa)
    wait_out(N_CHUNKS - 1, out_b, sem_ob)


@jax.jit
def _permute(x_flat, idx):
    kern = pl.kernel(
        _permute_body,
        out_type=jax.ShapeDtypeStruct((N_ROWS * N_COLS,), jnp.float32),
        mesh=plsc.VectorSubcoreMesh(core_axis_name="c", subcore_axis_name="s"),
        scratch_types=[
            pltpu.VMEM((N_COLS,), jnp.int32),
            pltpu.VMEM((CHUNK * N_COLS,), jnp.int32),
            pltpu.VMEM((CHUNK * N_COLS,), jnp.float32),
            pltpu.VMEM((CHUNK * N_COLS,), jnp.float32),
            pltpu.VMEM((CHUNK * N_COLS,), jnp.float32),
            pltpu.VMEM((CHUNK * N_COLS,), jnp.float32),
            pltpu.SemaphoreType.DMA,
            pltpu.SemaphoreType.DMA,
            pltpu.SemaphoreType.DMA,
            pltpu.SemaphoreType.DMA,
        ],
        compiler_params=pltpu.CompilerParams(needs_layout_passes=False),
    )
    return kern(x_flat, idx)


def kernel(x, perm, perm_inv, rev):
    idx = jnp.where(rev, perm_inv, perm).astype(jnp.int32)
    out_flat = _permute(x.reshape(-1), idx)
    return out_flat.reshape(N_ROWS, N_COLS)
